# baseline (speedup 1.0000x reference)
import functools

import jax
import jax.numpy as jnp
from jax import lax
from jax.experimental import pallas as pl
from jax.experimental.pallas import tpu as pltpu
from jax.experimental.pallas import tpu_sc as plsc

VOCAB = 1000000
EMBED = 64
NTOK = 200
BATCH = 4096

NUM_CORES = 2
NUM_SUBCORES = 16
NW = NUM_CORES * NUM_SUBCORES
BCOLS = BATCH // NW

VBLK = 128
NFULL = VOCAB // VBLK
TAIL = VOCAB - NFULL * VBLK
ORD_PER_W = NFULL // NW + 1
PROWS = VOCAB // 2


def _widx(k):
    return lax.iota(jnp.int32, 16) + 16 * k


@functools.lru_cache(maxsize=1)
def _build_kernels():
    mesh = plsc.VectorSubcoreMesh(core_axis_name="c", subcore_axis_name="s")
    params = pltpu.CompilerParams(use_tc_tiling_on_sc=True,
                                  needs_layout_passes=False)

    @functools.partial(
        pl.kernel,
        mesh=mesh,
        out_type=jax.ShapeDtypeStruct((PROWS, 128), jnp.float32),
        compiler_params=params,
        scratch_types=[
            pltpu.VMEM((EMBED, VBLK), jnp.float32),
            pltpu.VMEM((EMBED, VBLK), jnp.float32),
            pltpu.VMEM((VBLK // 2, 128), jnp.float32),
            pltpu.VMEM((VBLK // 2, 128), jnp.float32),
            pltpu.SemaphoreType.DMA,
            pltpu.SemaphoreType.DMA,
            pltpu.SemaphoreType.DMA,
            pltpu.SemaphoreType.DMA,
        ],
    )
    def _table_transpose(tt_hbm, tail_hbm, tp_hbm,
                         ebuf0, ebuf1, obuf0, obuf1,
                         lsem0, lsem1, ssem0, ssem1):
        ebufs = (ebuf0, ebuf1)
        obufs = (obuf0, obuf1)
        lsems = (lsem0, lsem1)
        ssems = (ssem0, ssem1)
        wid = lax.axis_index("s") * NUM_CORES + lax.axis_index("c")

        def blk_of(o):
            return jnp.minimum(wid + NW * o, NFULL - 1)

        def issue_load(o, p):
            pltpu.async_copy(
                tt_hbm.at[:, pl.ds(pl.multiple_of(blk_of(o) * VBLK, VBLK),
                                   VBLK)],
                ebufs[p], lsems[p])

        def wait_load(p):
            pltpu.make_async_copy(
                tt_hbm.at[:, pl.ds(0, VBLK)], ebufs[p], lsems[p]).wait()

        def issue_store(o, p):
            pltpu.async_copy(
                obufs[p],
                tp_hbm.at[pl.ds(pl.multiple_of(blk_of(o) * (VBLK // 2), 8),
                                VBLK // 2)],
                ssems[p])

        def wait_store(p):
            pltpu.make_async_copy(
                obufs[p],
                tp_hbm.at[pl.ds(0, VBLK // 2)], ssems[p]).wait()

        def transpose(p):
            def row(r, acc):
                c0 = jnp.broadcast_to(2 * r, (16,)).astype(jnp.int32)
                c1 = c0 + 1
                for k in range(EMBED // 16):
                    obufs[p][r, pl.ds(16 * k, 16)] = plsc.load_gather(
                        ebufs[p], [_widx(k), c0])
                    obufs[p][r, pl.ds(64 + 16 * k, 16)] = plsc.load_gather(
                        ebufs[p], [_widx(k), c1])
                return acc
            lax.fori_loop(0, VBLK // 2, row, 0)

        issue_load(0, 0)
        issue_load(1, 1)

        def body(i, carry):
            for p in range(2):
                o = i * 2 + p
                wait_load(p)

                @pl.when(i > 0)
                def _():
                    wait_store(p)

                transpose(p)
                issue_store(o, p)

                @pl.when(o + 2 < ORD_PER_W)
                def _():
                    issue_load(o + 2, p)
            return carry

        lax.fori_loop(0, ORD_PER_W // 2, body, 0)

        wait_load(0)
        wait_store(0)
        transpose(0)
        issue_store(ORD_PER_W - 1, 0)
        wait_store(1)

        @pl.when(wid == NW - 1)
        def _():
            pltpu.sync_copy(tail_hbm, ebuf1)

            def row(r, acc):
                c0 = jnp.broadcast_to(2 * r, (16,)).astype(jnp.int32)
                c1 = c0 + 1
                for k in range(EMBED // 16):
                    obuf1[r, pl.ds(16 * k, 16)] = plsc.load_gather(
                        ebuf1, [_widx(k), c0])
                    obuf1[r, pl.ds(64 + 16 * k, 16)] = plsc.load_gather(
                        ebuf1, [_widx(k), c1])
                return acc
            lax.fori_loop(0, TAIL // 2, row, 0)
            pltpu.sync_copy(
                obuf1.at[pl.ds(0, TAIL // 2)],
                tp_hbm.at[pl.ds(PROWS - TAIL // 2, TAIL // 2)])

        wait_store(0)

    @functools.partial(
        pl.kernel,
        mesh=mesh,
        out_type=jax.ShapeDtypeStruct((NTOK, EMBED, BATCH), jnp.float32),
        compiler_params=params,
        scratch_types=[
            pltpu.VMEM((NTOK, BCOLS), jnp.int32),
            pltpu.VMEM((NTOK, BCOLS), jnp.int32),
            pltpu.VMEM((NTOK, EMBED), jnp.float32),
            pltpu.VMEM((BCOLS, 128), jnp.float32),
            pltpu.VMEM((BCOLS, 128), jnp.float32),
            pltpu.VMEM((EMBED, BCOLS), jnp.float32),
            pltpu.VMEM((EMBED, BCOLS), jnp.float32),
            pltpu.SemaphoreType.DMA,
            pltpu.SemaphoreType.DMA,
            pltpu.SemaphoreType.DMA,
            pltpu.SemaphoreType.DMA,
        ],
    )
    def _embed_gather(tp_hbm, tokh_hbm, tokp_hbm, pos_hbm, out_hbm,
                      idx_v, par_v, pos_v, gbuf0, gbuf1, sbuf0, sbuf1,
                      gsem0, gsem1, ssem0, ssem1):
        gbufs = (gbuf0, gbuf1)
        sbufs = (sbuf0, sbuf1)
        gsems = (gsem0, gsem1)
        ssems = (ssem0, ssem1)
        wid = lax.axis_index("s") * NUM_CORES + lax.axis_index("c")
        b0 = pl.multiple_of(wid * BCOLS, BCOLS)

        pltpu.sync_copy(tokh_hbm.at[:, pl.ds(b0, BCOLS)], idx_v)
        pltpu.sync_copy(tokp_hbm.at[:, pl.ds(b0, BCOLS)], par_v)
        pltpu.sync_copy(pos_hbm, pos_v)

        def issue_gather(t, p):
            pltpu.async_copy(tp_hbm.at[idx_v.at[t]], gbufs[p], gsems[p])

        def wait_gather(p):
            pltpu.make_async_copy(
                tp_hbm.at[pl.ds(0, BCOLS)], gbufs[p], gsems[p]).wait()

        def issue_store(t, p):
            pltpu.async_copy(
                sbufs[p], out_hbm.at[t, :, pl.ds(b0, BCOLS)], ssems[p])

        def wait_store(p):
            pltpu.make_async_copy(
                sbufs[p],
                out_hbm.at[0, :, pl.ds(0, BCOLS)], ssems[p]).wait()

        def transpose_add(t, p):
            tt = jnp.broadcast_to(t, (16,)).astype(jnp.int32)
            par64 = [par_v[t, pl.ds(16 * k, 16)] * EMBED
                     for k in range(BCOLS // 16)]

            def erow(e, acc):
                ee = jnp.broadcast_to(e, (16,)).astype(jnp.int32)
                pe = plsc.load_gather(pos_v, [tt, ee])
                for k in range(BCOLS // 16):
                    sbufs[p][e, pl.ds(16 * k, 16)] = pe + plsc.load_gather(
                        gbufs[p], [_widx(k), ee + par64[k]])
                return acc
            lax.fori_loop(0, EMBED, erow, 0)

        issue_gather(0, 0)
        issue_gather(1, 1)

        def body(i, carry):
            for p in range(2):
                t = i * 2 + p
                wait_gather(p)

                @pl.when(i > 0)
                def _():
                    wait_store(p)

                transpose_add(t, p)
                issue_store(t, p)

                @pl.when(t + 2 < NTOK)
                def _():
                    issue_gather(t + 2, p)
            return carry

        lax.fori_loop(0, NTOK // 2, body, 0)
        wait_store(0)
        wait_store(1)

    return _table_transpose, _embed_gather


def kernel(tokens, token_embedding, position_embedding):
    k1, k2 = _build_kernels()
    tail = jnp.pad(token_embedding[VOCAB - TAIL:].T,
                   ((0, 0), (0, VBLK - TAIL)))
    staged = k1(token_embedding.T, tail)
    tokh = (tokens >> 1).T
    tokp = (tokens & 1).T
    out_t = k2(staged, tokh, tokp, position_embedding)
    return out_t.transpose(2, 0, 1)


# speedup vs baseline: 6.4812x; 6.4812x over previous
import functools

import jax
import jax.numpy as jnp
from jax import lax
from jax.experimental import pallas as pl
from jax.experimental.pallas import tpu as pltpu
from jax.experimental.pallas import tpu_sc as plsc

VOCAB = 1000000
EMBED = 64
NTOK = 200
BATCH = 4096

NUM_CORES = 2
NUM_SUBCORES = 16
NW = NUM_CORES * NUM_SUBCORES
BCOLS = BATCH // NW

VBLK = 128
NFULL = VOCAB // VBLK
TAIL = VOCAB - NFULL * VBLK
ORD_PER_W = NFULL // NW + 1
PROWS = VOCAB // 2


def _widx(k):
    return lax.iota(jnp.int32, 16) + 16 * k


@functools.lru_cache(maxsize=1)
def _build_kernels():
    mesh = plsc.VectorSubcoreMesh(core_axis_name="c", subcore_axis_name="s")
    params = pltpu.CompilerParams(use_tc_tiling_on_sc=True,
                                  needs_layout_passes=False)

    @functools.partial(
        pl.kernel,
        mesh=mesh,
        out_type=jax.ShapeDtypeStruct((PROWS, 128), jnp.float32),
        compiler_params=params,
        scratch_types=[
            pltpu.VMEM((EMBED, VBLK), jnp.float32),
            pltpu.VMEM((EMBED, VBLK), jnp.float32),
            pltpu.VMEM((VBLK // 2, 128), jnp.float32),
            pltpu.VMEM((VBLK // 2, 128), jnp.float32),
            pltpu.SemaphoreType.DMA,
            pltpu.SemaphoreType.DMA,
            pltpu.SemaphoreType.DMA,
            pltpu.SemaphoreType.DMA,
        ],
    )
    def _table_transpose(tt_hbm, tail_hbm, tp_hbm,
                         ebuf0, ebuf1, obuf0, obuf1,
                         lsem0, lsem1, ssem0, ssem1):
        ebufs = (ebuf0, ebuf1)
        obufs = (obuf0, obuf1)
        lsems = (lsem0, lsem1)
        ssems = (ssem0, ssem1)
        wid = lax.axis_index("s") * NUM_CORES + lax.axis_index("c")

        def blk_of(o):
            return jnp.minimum(wid + NW * o, NFULL - 1)

        def issue_load(o, p):
            pltpu.async_copy(
                tt_hbm.at[:, pl.ds(pl.multiple_of(blk_of(o) * VBLK, VBLK),
                                   VBLK)],
                ebufs[p], lsems[p])

        def wait_load(p):
            pltpu.make_async_copy(
                tt_hbm.at[:, pl.ds(0, VBLK)], ebufs[p], lsems[p]).wait()

        def issue_store(o, p):
            pltpu.async_copy(
                obufs[p],
                tp_hbm.at[pl.ds(pl.multiple_of(blk_of(o) * (VBLK // 2), 8),
                                VBLK // 2)],
                ssems[p])

        def wait_store(p):
            pltpu.make_async_copy(
                obufs[p],
                tp_hbm.at[pl.ds(0, VBLK // 2)], ssems[p]).wait()

        def transpose(p):
            @functools.partial(plsc.parallel_loop, 0, VBLK // 2, unroll=4)
            def _row(r):
                c0 = jnp.broadcast_to(2 * r, (16,)).astype(jnp.int32)
                c1 = c0 + 1
                for k in range(EMBED // 16):
                    obufs[p][r, pl.ds(16 * k, 16)] = plsc.load_gather(
                        ebufs[p], [_widx(k), c0])
                    obufs[p][r, pl.ds(64 + 16 * k, 16)] = plsc.load_gather(
                        ebufs[p], [_widx(k), c1])

        issue_load(0, 0)
        issue_load(1, 1)

        def body(i, carry):
            for p in range(2):
                o = i * 2 + p
                wait_load(p)

                @pl.when(i > 0)
                def _():
                    wait_store(p)

                transpose(p)
                issue_store(o, p)

                @pl.when(o + 2 < ORD_PER_W)
                def _():
                    issue_load(o + 2, p)
            return carry

        lax.fori_loop(0, ORD_PER_W // 2, body, 0)

        wait_load(0)
        wait_store(0)
        transpose(0)
        issue_store(ORD_PER_W - 1, 0)
        wait_store(1)

        @pl.when(wid == NW - 1)
        def _():
            pltpu.sync_copy(tail_hbm, ebuf1)

            def row(r, acc):
                c0 = jnp.broadcast_to(2 * r, (16,)).astype(jnp.int32)
                c1 = c0 + 1
                for k in range(EMBED // 16):
                    obuf1[r, pl.ds(16 * k, 16)] = plsc.load_gather(
                        ebuf1, [_widx(k), c0])
                    obuf1[r, pl.ds(64 + 16 * k, 16)] = plsc.load_gather(
                        ebuf1, [_widx(k), c1])
                return acc
            lax.fori_loop(0, TAIL // 2, row, 0)
            pltpu.sync_copy(
                obuf1.at[pl.ds(0, TAIL // 2)],
                tp_hbm.at[pl.ds(PROWS - TAIL // 2, TAIL // 2)])

        wait_store(0)

    @functools.partial(
        pl.kernel,
        mesh=mesh,
        out_type=jax.ShapeDtypeStruct((NTOK, EMBED, BATCH), jnp.float32),
        compiler_params=params,
        scratch_types=[
            pltpu.VMEM((NTOK, BCOLS), jnp.int32),
            pltpu.VMEM((NTOK, BCOLS), jnp.int32),
            pltpu.VMEM((NTOK, EMBED), jnp.float32),
            pltpu.VMEM((BCOLS, 128), jnp.float32),
            pltpu.VMEM((BCOLS, 128), jnp.float32),
            pltpu.VMEM((EMBED, BCOLS), jnp.float32),
            pltpu.VMEM((EMBED, BCOLS), jnp.float32),
            pltpu.SemaphoreType.DMA,
            pltpu.SemaphoreType.DMA,
            pltpu.SemaphoreType.DMA,
            pltpu.SemaphoreType.DMA,
        ],
    )
    def _embed_gather(tp_hbm, tokh_hbm, tokp_hbm, pos_hbm, out_hbm,
                      idx_v, par_v, pos_v, gbuf0, gbuf1, sbuf0, sbuf1,
                      gsem0, gsem1, ssem0, ssem1):
        gbufs = (gbuf0, gbuf1)
        sbufs = (sbuf0, sbuf1)
        gsems = (gsem0, gsem1)
        ssems = (ssem0, ssem1)
        wid = lax.axis_index("s") * NUM_CORES + lax.axis_index("c")
        b0 = pl.multiple_of(wid * BCOLS, BCOLS)

        pltpu.sync_copy(tokh_hbm.at[:, pl.ds(b0, BCOLS)], idx_v)
        pltpu.sync_copy(tokp_hbm.at[:, pl.ds(b0, BCOLS)], par_v)
        pltpu.sync_copy(pos_hbm, pos_v)

        def issue_gather(t, p):
            pltpu.async_copy(tp_hbm.at[idx_v.at[t]], gbufs[p], gsems[p])

        def wait_gather(p):
            pltpu.make_async_copy(
                tp_hbm.at[pl.ds(0, BCOLS)], gbufs[p], gsems[p]).wait()

        def issue_store(t, p):
            pltpu.async_copy(
                sbufs[p], out_hbm.at[t, :, pl.ds(b0, BCOLS)], ssems[p])

        def wait_store(p):
            pltpu.make_async_copy(
                sbufs[p],
                out_hbm.at[0, :, pl.ds(0, BCOLS)], ssems[p]).wait()

        def transpose_add(t, p):
            tt = jnp.broadcast_to(t, (16,)).astype(jnp.int32)
            par64 = [par_v[t, pl.ds(16 * k, 16)] * EMBED
                     for k in range(BCOLS // 16)]

            @functools.partial(plsc.parallel_loop, 0, EMBED, unroll=4)
            def _erow(e):
                ee = jnp.broadcast_to(e, (16,)).astype(jnp.int32)
                pe = plsc.load_gather(pos_v, [tt, ee])
                for k in range(BCOLS // 16):
                    sbufs[p][e, pl.ds(16 * k, 16)] = pe + plsc.load_gather(
                        gbufs[p], [_widx(k), ee + par64[k]])

        issue_gather(0, 0)
        issue_gather(1, 1)

        def body(i, carry):
            for p in range(2):
                t = i * 2 + p
                wait_gather(p)

                @pl.when(i > 0)
                def _():
                    wait_store(p)

                transpose_add(t, p)
                issue_store(t, p)

                @pl.when(t + 2 < NTOK)
                def _():
                    issue_gather(t + 2, p)
            return carry

        lax.fori_loop(0, NTOK // 2, body, 0)
        wait_store(0)
        wait_store(1)

    return _table_transpose, _embed_gather


def kernel(tokens, token_embedding, position_embedding):
    k1, k2 = _build_kernels()
    tail = jnp.pad(token_embedding[VOCAB - TAIL:].T,
                   ((0, 0), (0, VBLK - TAIL)))
    staged = k1(token_embedding.T, tail)
    tokh = (tokens >> 1).T
    tokp = (tokens & 1).T
    out_t = k2(staged, tokh, tokp, position_embedding)
    return out_t.transpose(2, 0, 1)
